# Initial kernel scaffold; baseline (speedup 1.0000x reference)
import jax
import jax.numpy as jnp
from jax.experimental import pallas as pl


def kernel(x, t, decs, paths, lens):
    raise NotImplementedError("write your pallas kernel here")


# speedup vs baseline: 12.3756x; 12.3756x over previous
import functools

import jax
import jax.numpy as jnp
from jax import lax
from jax.experimental import pallas as pl
from jax.experimental.pallas import tpu as pltpu
from jax.experimental.pallas import tpu_sc as plsc

N_VOCAB = 100000
N_DEC = N_VOCAB - 1
MAX_PATH = 32
D = 512
B = 8192

NC = 2
NS = 16
LANES = 16
NW = NC * NS
TPW = B // NW
NCHUNK = D // LANES
AUXW = 128
SENTINEL = 1e4


def _sc_scores():
    mesh = plsc.VectorSubcoreMesh(core_axis_name="c", subcore_axis_name="s")

    @functools.partial(
        pl.kernel,
        out_type=jax.ShapeDtypeStruct((B, MAX_PATH), jnp.float32),
        mesh=mesh,
        compiler_params=pltpu.CompilerParams(needs_layout_passes=False),
        scratch_types=[
            pltpu.VMEM((TPW,), jnp.int32),
            pltpu.VMEM((TPW, AUXW), jnp.int32),
            pltpu.VMEM((2, 1, D), jnp.float32),
            pltpu.VMEM((2, MAX_PATH, D), jnp.float32),
            pltpu.VMEM((LANES * LANES,), jnp.float32),
            pltpu.VMEM((TPW, MAX_PATH), jnp.float32),
            pltpu.SemaphoreType.DMA,
            pltpu.SemaphoreType.DMA,
            pltpu.SemaphoreType.DMA,
            pltpu.SemaphoreType.DMA,
        ],
    )
    def sc_kernel(x_hbm, t_hbm, aux_hbm, decs_hbm,
                  scores_hbm,
                  t_v, aux_v, xrow_v, rows_v, accs_v, sb_v,
                  semr0, semr1, semx0, semx1):
        wid = lax.axis_index("s") * NC + lax.axis_index("c")
        base = wid * TPW

        pltpu.sync_copy(t_hbm.at[pl.ds(base, TPW)], t_v)
        for g in range(TPW // 128):
            sl = pl.ds(g * 128, 128)
            pltpu.async_copy(aux_hbm.at[t_v.at[sl]], aux_v.at[sl], semr0).wait()

        iota16 = lax.iota(jnp.int32, LANES)
        semr = (semr0, semr1)
        semx = (semx0, semx1)

        def fire(i, slot):
            pltpu.async_copy(decs_hbm.at[aux_v.at[i, pl.ds(0, MAX_PATH)]],
                             rows_v.at[slot], semr[slot])
            pltpu.async_copy(x_hbm.at[pl.ds(base + i, 1)],
                             xrow_v.at[slot], semx[slot])

        def wait(i, slot):
            pltpu.make_async_copy(decs_hbm.at[aux_v.at[i, pl.ds(0, MAX_PATH)]],
                                  rows_v.at[slot], semr[slot]).wait()
            pltpu.make_async_copy(x_hbm.at[pl.ds(base + i, 1)],
                                  xrow_v.at[slot], semx[slot]).wait()

        def compute(i, slot):
            rows = rows_v.at[slot]
            len_bc = aux_v[i, pl.ds(MAX_PATH, LANES)]
            zero = jnp.zeros((LANES,), jnp.float32)
            for jg in range(MAX_PATH // LANES):
                def cb_body(cb, accs):
                    out = list(accs)
                    for u in range(4):
                        off = (cb * 4 + u) * LANES
                        xc = xrow_v[slot, 0, pl.ds(off, LANES)]
                        for j16 in range(LANES):
                            j = jg * LANES + j16
                            out[j16] = out[j16] + rows[j, pl.ds(off, LANES)] * xc
                    return tuple(out)

                accs = lax.fori_loop(0, NCHUNK // 4, cb_body, (zero,) * LANES)
                for j16 in range(LANES):
                    accs_v[pl.ds(j16 * LANES, LANES)] = accs[j16]
                row_base = iota16 * LANES
                s0 = plsc.load_gather(accs_v, [row_base])
                s1 = plsc.load_gather(accs_v, [row_base + 1])
                for k in range(2, LANES, 2):
                    s0 = s0 + plsc.load_gather(accs_v, [row_base + k])
                    s1 = s1 + plsc.load_gather(accs_v, [row_base + k + 1])
                mask = (iota16 + jg * LANES) < len_bc
                s = jnp.where(mask, s0 + s1, SENTINEL)
                sb_v[i, pl.ds(jg * LANES, LANES)] = s

        fire(0, 0)

        def pair_body(g, _):
            i0 = 2 * g
            fire(i0 + 1, 1)
            wait(i0, 0)
            compute(i0, 0)

            @pl.when(g < TPW // 2 - 1)
            def _():
                fire(i0 + 2, 0)

            wait(i0 + 1, 1)
            compute(i0 + 1, 1)
            return 0

        lax.fori_loop(0, TPW // 2, pair_body, 0)
        pltpu.sync_copy(sb_v, scores_hbm.at[pl.ds(base, TPW)])

    return sc_kernel


_SC_SCORES = _sc_scores()


def _tc_loss_body(s_ref, o_ref):
    ls = jax.nn.log_sigmoid(s_ref[...])
    o_ref[...] = jnp.reshape(-jnp.sum(ls) / B, (1, 1))


_TC_LOSS = pl.pallas_call(
    _tc_loss_body,
    out_shape=jax.ShapeDtypeStruct((1, 1), jnp.float32),
)


def kernel(x, t, decs, paths, lens):
    t = t.astype(jnp.int32)
    aux = jnp.concatenate(
        [
            paths.astype(jnp.int32),
            jnp.broadcast_to(lens.astype(jnp.int32)[:, None], (N_VOCAB, LANES)),
            jnp.zeros((N_VOCAB, AUXW - MAX_PATH - LANES), jnp.int32),
        ],
        axis=1,
    )
    scores = _SC_SCORES(x, t, aux, decs)
    loss = _TC_LOSS(scores)
    return loss[0, 0]


# speedup vs baseline: 13.0669x; 1.0559x over previous
import functools

import jax
import jax.numpy as jnp
from jax import lax
from jax.experimental import pallas as pl
from jax.experimental.pallas import tpu as pltpu
from jax.experimental.pallas import tpu_sc as plsc

N_VOCAB = 100000
N_DEC = N_VOCAB - 1
MAX_PATH = 32
D = 512
B = 8192

NC = 2
NS = 16
LANES = 16
NW = NC * NS
TPW = B // NW
NCHUNK = D // LANES
AUXW = 128
SENTINEL = 1e4
NSLOT = 4
NAUX = 8
XBLK = 16


def _sc_scores():
    mesh = plsc.VectorSubcoreMesh(core_axis_name="c", subcore_axis_name="s")

    @functools.partial(
        pl.kernel,
        out_type=jax.ShapeDtypeStruct((B, MAX_PATH), jnp.float32),
        mesh=mesh,
        compiler_params=pltpu.CompilerParams(needs_layout_passes=False),
        scratch_types=[
            pltpu.VMEM((TPW + LANES,), jnp.int32),
            pltpu.VMEM((NAUX, AUXW), jnp.int32),
            pltpu.VMEM((2, XBLK, D), jnp.float32),
            pltpu.VMEM((NSLOT, MAX_PATH, D), jnp.float32),
            pltpu.VMEM((LANES * LANES,), jnp.float32),
            pltpu.VMEM((TPW, MAX_PATH), jnp.float32),
            pltpu.SemaphoreType.DMA((NAUX,)),
            [pltpu.SemaphoreType.DMA] * NSLOT,
            [pltpu.SemaphoreType.DMA] * NSLOT,
            pltpu.SemaphoreType.DMA((2,)),
        ],
    )
    def sc_kernel(x_hbm, t_hbm, aux_hbm, decs_hbm,
                  scores_hbm,
                  t_v, auxr_v, xblk_v, rows_v, accs_v, sb_v,
                  sema, semr, semh, semx):
        wid = lax.axis_index("s") * NC + lax.axis_index("c")
        base = wid * TPW

        pltpu.sync_copy(t_hbm.at[pl.ds(base, TPW)], t_v.at[pl.ds(0, TPW)])

        iota16 = lax.iota(jnp.int32, LANES)
        sent16 = jnp.full((LANES,), SENTINEL, jnp.float32)

        def tok(i):
            return t_v[pl.ds(i, LANES)][0]

        def afire(i):
            pltpu.async_copy(aux_hbm.at[pl.ds(tok(i), 1)],
                             auxr_v.at[pl.ds(i % NAUX, 1)],
                             sema.at[i % NAUX])

        def await_(i):
            pltpu.make_async_copy(aux_hbm.at[pl.ds(tok(i), 1)],
                                  auxr_v.at[pl.ds(i % NAUX, 1)],
                                  sema.at[i % NAUX]).wait()

        def lenvec(i):
            return auxr_v[i % NAUX, pl.ds(MAX_PATH, LANES)]

        def fire(i, slot):
            pltpu.async_copy(decs_hbm.at[auxr_v.at[i % NAUX, pl.ds(0, LANES)]],
                             rows_v.at[slot, pl.ds(0, LANES)], semr[slot])

            @pl.when(lenvec(i)[0] > LANES)
            def _():
                pltpu.async_copy(
                    decs_hbm.at[auxr_v.at[i % NAUX, pl.ds(LANES, LANES)]],
                    rows_v.at[slot, pl.ds(LANES, LANES)], semh[slot])

        def wait(i, slot):
            pltpu.make_async_copy(
                decs_hbm.at[auxr_v.at[i % NAUX, pl.ds(0, LANES)]],
                rows_v.at[slot, pl.ds(0, LANES)], semr[slot]).wait()

            @pl.when(lenvec(i)[0] > LANES)
            def _():
                pltpu.make_async_copy(
                    decs_hbm.at[auxr_v.at[i % NAUX, pl.ds(LANES, LANES)]],
                    rows_v.at[slot, pl.ds(LANES, LANES)], semh[slot]).wait()

        def xfire(c, xslot):
            pltpu.async_copy(x_hbm.at[pl.ds(base + c * XBLK, XBLK)],
                             xblk_v.at[xslot], semx.at[xslot])

        def xwait(c, xslot):
            pltpu.make_async_copy(x_hbm.at[pl.ds(base + c * XBLK, XBLK)],
                                  xblk_v.at[xslot], semx.at[xslot]).wait()

        def group(i, slot, jg):
            rows = rows_v.at[slot]
            len_bc = lenvec(i)
            zero = jnp.zeros((LANES,), jnp.float32)
            xrow = (i // XBLK) % 2
            xtok = i % XBLK

            def cb_body(cb, accs):
                out = list(accs)
                for u in range(4):
                    off = (cb * 4 + u) * LANES
                    xc = xblk_v[xrow, xtok, pl.ds(off, LANES)]
                    for j16 in range(LANES):
                        j = jg * LANES + j16
                        out[j16] = out[j16] + rows[j, pl.ds(off, LANES)] * xc
                return tuple(out)

            accs = lax.fori_loop(0, NCHUNK // 4, cb_body, (zero,) * LANES)
            for j16 in range(LANES):
                accs_v[pl.ds(j16 * LANES, LANES)] = accs[j16]
            row_base = iota16 * LANES
            s0 = plsc.load_gather(accs_v, [row_base])
            s1 = plsc.load_gather(accs_v, [row_base + 1])
            for k in range(2, LANES, 2):
                s0 = s0 + plsc.load_gather(accs_v, [row_base + k])
                s1 = s1 + plsc.load_gather(accs_v, [row_base + k + 1])
            mask = (iota16 + jg * LANES) < len_bc
            s = jnp.where(mask, s0 + s1, SENTINEL)
            sb_v[i, pl.ds(jg * LANES, LANES)] = s

        def compute(i, slot):
            group(i, slot, 0)
            sb_v[i, pl.ds(LANES, LANES)] = sent16

            @pl.when(lenvec(i)[0] > LANES)
            def _():
                group(i, slot, 1)

        xfire(0, 0)
        xwait(0, 0)
        for i in range(NAUX - 1):
            afire(i)
        for s in range(NSLOT - 1):
            await_(s)
            fire(s, s)

        def quad_body(g, _):
            i0 = NSLOT * g

            @pl.when((g % (XBLK // NSLOT) == 1)
                     & (i0 // XBLK + 1 < TPW // XBLK))
            def _():
                xfire(i0 // XBLK + 1, (i0 // XBLK + 1) % 2)

            @pl.when((g % (XBLK // NSLOT) == 0) & (g > 0))
            def _():
                xwait(i0 // XBLK, (i0 // XBLK) % 2)

            for u in range(NSLOT):
                i = i0 + u

                @pl.when(i + NSLOT - 1 < TPW)
                def _():
                    await_(i + NSLOT - 1)
                    fire(i + NSLOT - 1, (u + NSLOT - 1) % NSLOT)

                @pl.when(i + NAUX - 1 < TPW)
                def _():
                    afire(i + NAUX - 1)

                wait(i, u)
                compute(i, u)
            return 0

        lax.fori_loop(0, TPW // NSLOT, quad_body, 0)
        pltpu.sync_copy(sb_v, scores_hbm.at[pl.ds(base, TPW)])

    return sc_kernel


_SC_SCORES = _sc_scores()


_AUX_BK = 2000


def _aux_body(p_ref, l_ref, o_ref):
    p = p_ref[...]
    l = l_ref[0]
    lt = jnp.transpose(jnp.broadcast_to(l, (LANES, _AUX_BK)), (1, 0))
    o_ref[...] = jnp.concatenate(
        [
            p,
            lt,
            jnp.zeros((_AUX_BK, AUXW - MAX_PATH - LANES), jnp.int32),
        ],
        axis=1,
    )


_AUX_BUILD = pl.pallas_call(
    _aux_body,
    grid=(N_VOCAB // _AUX_BK,),
    in_specs=[
        pl.BlockSpec((_AUX_BK, MAX_PATH), lambda i: (i, 0)),
        pl.BlockSpec((1, 1, _AUX_BK), lambda i: (i, 0, 0)),
    ],
    out_specs=pl.BlockSpec((_AUX_BK, AUXW), lambda i: (i, 0)),
    out_shape=jax.ShapeDtypeStruct((N_VOCAB, AUXW), jnp.int32),
)


def _tc_loss_body(s_ref, o_ref):
    ls = jax.nn.log_sigmoid(s_ref[...])
    o_ref[...] = jnp.reshape(-jnp.sum(ls) / B, (1, 1))


_TC_LOSS = pl.pallas_call(
    _tc_loss_body,
    out_shape=jax.ShapeDtypeStruct((1, 1), jnp.float32),
)


def kernel(x, t, decs, paths, lens):
    t = t.astype(jnp.int32)
    aux = _AUX_BUILD(paths.astype(jnp.int32),
                     lens.astype(jnp.int32).reshape(N_VOCAB // _AUX_BK, 1,
                                                    _AUX_BK))
    scores = _SC_SCORES(x, t, aux, decs)
    loss = _TC_LOSS(scores)
    return loss[0, 0]
